# baseline (speedup 1.0000x reference)
import functools

import jax
import jax.numpy as jnp
from jax import lax
from jax.experimental import pallas as pl
from jax.experimental.pallas import tpu as pltpu
from jax.experimental.pallas import tpu_sc as plsc

NC = 2
NS = 16
NW = NC * NS

BBLK = 128


@functools.lru_cache(maxsize=None)
def _build_gather(vocab, emb, l_seq, batch):
    nblk = batch // BBLK // NW
    bspan = nblk * BBLK
    nblocks = l_seq * nblk
    assert nblocks % 2 == 0

    mesh = plsc.VectorSubcoreMesh(core_axis_name="c", subcore_axis_name="s")

    @functools.partial(
        pl.kernel,
        out_type=jax.ShapeDtypeStruct((l_seq, emb, batch), jnp.float32),
        mesh=mesh,
        scratch_types=[
            pltpu.VMEM((l_seq, bspan), jnp.int32),
            pltpu.VMEM((2, BBLK, 128), jnp.float32),
            pltpu.VMEM((2, emb, BBLK), jnp.float32),
            [pltpu.SemaphoreType.DMA] * 2,
            [pltpu.SemaphoreType.DMA] * 2,
        ],
        compiler_params=pltpu.CompilerParams(use_tc_tiling_on_sc=True,
                                             needs_layout_passes=False),
    )
    def k1(t128_hbm, xt_hbm, out_hbm, idx_v, g_v, t_v, gsems, wsems):
        wid = lax.axis_index("s") * NC + lax.axis_index("c")
        b0 = wid * bspan

        pltpu.sync_copy(xt_hbm.at[:, pl.ds(b0 * 1, bspan)], idx_v)

        iota = lax.iota(jnp.int32, 16)
        row_idx = [iota + (t0 * 16) for t0 in range(8)]

        def fire_gather(p, i):
            l, k = i // nblk, i % nblk
            pltpu.async_copy(
                t128_hbm.at[idx_v.at[l, pl.ds(k * BBLK, BBLK)]],
                g_v.at[p], gsems[p])

        def wait_gather(p):
            pltpu.make_async_copy(
                t128_hbm.at[idx_v.at[0, pl.ds(0, BBLK)]],
                g_v.at[p], gsems[p]).wait()

        def fire_write(p, i):
            l, k = i // nblk, i % nblk
            pltpu.async_copy(
                t_v.at[p],
                out_hbm.at[l, :, pl.ds(b0 + k * BBLK, BBLK)], wsems[p])

        def wait_write(p):
            pltpu.make_async_copy(
                t_v.at[p],
                out_hbm.at[0, :, pl.ds(0, BBLK)], wsems[p]).wait()

        def transpose(p):
            src = g_v.at[p]
            dst = t_v.at[p]
            for e in range(emb):
                col = jnp.full((16,), e, dtype=jnp.int32)
                for t0 in range(8):
                    v = plsc.load_gather(src, [row_idx[t0], col])
                    dst[e, pl.ds(t0 * 16, 16)] = v

        fire_gather(0, 0)
        fire_gather(1, 1)

        def outer(c, carry):
            for p in range(2):
                i = c * 2 + p
                wait_gather(p)

                @pl.when(c > 0)
                def _(p=p):
                    wait_write(p)

                transpose(p)
                fire_write(p, i)

                @pl.when(i + 2 < nblocks)
                def _(p=p, i=i):
                    fire_gather(p, i + 2)
            return carry

        lax.fori_loop(0, nblocks // 2, outer, 0, unroll=False)
        wait_write(0)
        wait_write(1)

    return k1


def kernel(x, TokenEmbeddings):
    batch, l_seq = x.shape
    vocab, emb = TokenEmbeddings.shape
    t128 = jnp.pad(TokenEmbeddings, ((0, 0), (0, 128 - emb)))
    xt = x.T.astype(jnp.int32)
    out3 = _build_gather(vocab, emb, l_seq, batch)(t128, xt)
    return jnp.transpose(out3, (2, 0, 1))


# speedup vs baseline: 1.2707x; 1.2707x over previous
import functools

import jax
import jax.numpy as jnp
from jax import lax
from jax.experimental import pallas as pl
from jax.experimental.pallas import tpu as pltpu
from jax.experimental.pallas import tpu_sc as plsc

NC = 2
NS = 16
NW = NC * NS

BBLK = 128


@functools.lru_cache(maxsize=None)
def _build_gather(vocab, emb, l_seq, batch):
    nblk = batch // BBLK // NW
    bspan = nblk * BBLK
    nblocks = l_seq * nblk
    assert nblocks % 2 == 0

    mesh = plsc.VectorSubcoreMesh(core_axis_name="c", subcore_axis_name="s")

    @functools.partial(
        pl.kernel,
        out_type=jax.ShapeDtypeStruct((l_seq, emb, batch), jnp.float32),
        mesh=mesh,
        scratch_types=[
            pltpu.VMEM((l_seq, bspan), jnp.int32),
            pltpu.VMEM((2, BBLK, 128), jnp.float32),
            pltpu.VMEM((2, emb, BBLK), jnp.float32),
            [pltpu.SemaphoreType.DMA] * 2,
            [pltpu.SemaphoreType.DMA] * 2,
        ],
        compiler_params=pltpu.CompilerParams(use_tc_tiling_on_sc=True,
                                             needs_layout_passes=False),
    )
    def k1(t128_hbm, xt_hbm, out_hbm, idx_v, g_v, t_v, gsems, wsems):
        wid = lax.axis_index("s") * NC + lax.axis_index("c")
        b0 = wid * bspan

        pltpu.sync_copy(xt_hbm.at[:, pl.ds(b0 * 1, bspan)], idx_v)

        iota = lax.iota(jnp.int32, 16)
        e_idx = [iota + e0 for e0 in range(0, emb, 16)]

        def fire_gather(p, i):
            l, k = i // nblk, i % nblk
            pltpu.async_copy(
                t128_hbm.at[idx_v.at[l, pl.ds(k * BBLK, BBLK)]],
                g_v.at[p], gsems[p])

        def wait_gather(p):
            pltpu.make_async_copy(
                t128_hbm.at[idx_v.at[0, pl.ds(0, BBLK)]],
                g_v.at[p], gsems[p]).wait()

        def fire_write(p, i):
            l, k = i // nblk, i % nblk
            pltpu.async_copy(
                t_v.at[p],
                out_hbm.at[l, :, pl.ds(b0 + k * BBLK, BBLK)], wsems[p])

        def wait_write(p):
            pltpu.make_async_copy(
                t_v.at[p],
                out_hbm.at[0, :, pl.ds(0, BBLK)], wsems[p]).wait()

        def transpose(p):
            src = g_v.at[p]
            dst = t_v.at[p]
            for t in range(BBLK):
                t_s = jnp.full((16,), t, dtype=jnp.int32)
                for j in range(emb // 16):
                    v = src[t, pl.ds(j * 16, 16)]
                    plsc.store_scatter(dst, [e_idx[j], t_s], v)

        fire_gather(0, 0)
        fire_gather(1, 1)

        def outer(c, carry):
            for p in range(2):
                i = c * 2 + p
                wait_gather(p)

                @pl.when(c > 0)
                def _(p=p):
                    wait_write(p)

                transpose(p)
                fire_write(p, i)

                @pl.when(i + 2 < nblocks)
                def _(p=p, i=i):
                    fire_gather(p, i + 2)
            return carry

        lax.fori_loop(0, nblocks // 2, outer, 0, unroll=False)
        wait_write(0)
        wait_write(1)

    return k1


def kernel(x, TokenEmbeddings):
    batch, l_seq = x.shape
    vocab, emb = TokenEmbeddings.shape
    t128 = jnp.pad(TokenEmbeddings, ((0, 0), (0, 128 - emb)))
    xt = x.T.astype(jnp.int32)
    out3 = _build_gather(vocab, emb, l_seq, batch)(t128, xt)
    return jnp.transpose(out3, (2, 0, 1))


# speedup vs baseline: 1.4048x; 1.1056x over previous
import functools

import jax
import jax.numpy as jnp
from jax import lax
from jax.experimental import pallas as pl
from jax.experimental.pallas import tpu as pltpu
from jax.experimental.pallas import tpu_sc as plsc

NC = 2
NS = 16
NW = NC * NS

BBLK = 128
NBUF = 4


@functools.lru_cache(maxsize=None)
def _build_gather(vocab, l_seq, batch):
    nblk = batch // BBLK // NW
    bspan = nblk * BBLK
    nblocks = l_seq * nblk
    assert nblocks % NBUF == 0

    mesh = plsc.VectorSubcoreMesh(core_axis_name="c", subcore_axis_name="s")

    @functools.partial(
        pl.kernel,
        out_type=jax.ShapeDtypeStruct((l_seq, batch, 128), jnp.float32),
        mesh=mesh,
        scratch_types=[
            pltpu.VMEM((l_seq, bspan), jnp.int32),
            pltpu.VMEM((NBUF, BBLK, 128), jnp.float32),
            [pltpu.SemaphoreType.DMA] * NBUF,
            [pltpu.SemaphoreType.DMA] * NBUF,
        ],
        compiler_params=pltpu.CompilerParams(use_tc_tiling_on_sc=True,
                                             needs_layout_passes=False),
    )
    def k1(t128_hbm, xt_hbm, out_hbm, idx_v, g_v, gsems, wsems):
        wid = lax.axis_index("s") * NC + lax.axis_index("c")
        b0 = wid * bspan

        pltpu.sync_copy(xt_hbm.at[:, pl.ds(b0, bspan)], idx_v)

        def fire_gather(p, i):
            l, k = i // nblk, i % nblk
            pltpu.async_copy(
                t128_hbm.at[idx_v.at[l, pl.ds(k * BBLK, BBLK)]],
                g_v.at[p], gsems[p])

        def wait_gather(p):
            pltpu.make_async_copy(
                t128_hbm.at[idx_v.at[0, pl.ds(0, BBLK)]],
                g_v.at[p], gsems[p]).wait()

        def fire_write(p, i):
            l, k = i // nblk, i % nblk
            pltpu.async_copy(
                g_v.at[p],
                out_hbm.at[l, pl.ds(b0 + k * BBLK, BBLK), :], wsems[p])

        def wait_write(p):
            pltpu.make_async_copy(
                g_v.at[p],
                out_hbm.at[0, pl.ds(0, BBLK), :], wsems[p]).wait()

        def outer(c, carry):
            for p in range(NBUF):
                @pl.when(c > 0)
                def _(p=p):
                    wait_write(p)

                fire_gather(p, c * NBUF + p)
            for p in range(NBUF):
                wait_gather(p)
                fire_write(p, c * NBUF + p)
            return carry

        lax.fori_loop(0, nblocks // NBUF, outer, 0, unroll=False)
        for p in range(NBUF):
            wait_write(p)

    return k1


@functools.lru_cache(maxsize=None)
def _build_transpose(emb, l_seq, batch):
    tblk = 1024

    def k2(in_ref, out_ref):
        out_ref[0] = in_ref[0, :, :emb].T

    return pl.pallas_call(
        k2,
        grid=(l_seq, batch // tblk),
        in_specs=[pl.BlockSpec((1, tblk, 128), lambda l, b: (l, b, 0))],
        out_specs=pl.BlockSpec((1, emb, tblk), lambda l, b: (l, 0, b)),
        out_shape=jax.ShapeDtypeStruct((l_seq, emb, batch), jnp.float32),
    )


def kernel(x, TokenEmbeddings):
    batch, l_seq = x.shape
    vocab, emb = TokenEmbeddings.shape
    t128 = jnp.pad(TokenEmbeddings, ((0, 0), (0, 128 - emb)))
    xt = x.T.astype(jnp.int32)
    mid = _build_gather(vocab, l_seq, batch)(t128, xt)
    out3 = _build_transpose(emb, l_seq, batch)(mid)
    return jnp.transpose(out3, (2, 0, 1))
